# baseline (speedup 1.0000x reference)
import functools

import jax
import jax.numpy as jnp
from jax import lax
from jax.experimental import pallas as pl
from jax.experimental.pallas import tpu as pltpu
from jax.experimental.pallas import tpu_sc as plsc

_B = 32
_C = 192
_H = 56
_W = 56
_P = _B * _H * _W

_info = plsc.get_sparse_core_info()
_NC = _info.num_cores
_NS = _info.num_subcores
_NW = _NC * _NS
_L = _info.num_lanes
_NBLK = _C // _L
_CHUNK = 112
_PPW = _P // _NW
_NCHUNK = _PPW // _CHUNK

_mesh = plsc.VectorSubcoreMesh(core_axis_name="c", subcore_axis_name="s")


@functools.partial(
    pl.kernel,
    mesh=_mesh,
    compiler_params=pltpu.CompilerParams(needs_layout_passes=False),
    out_type=jax.ShapeDtypeStruct((_P, _C), jnp.float32),
    scratch_types=[
        pltpu.VMEM((_C,), jnp.int32),
        pltpu.VMEM((_CHUNK, _C), jnp.float32),
        pltpu.VMEM((_CHUNK, _C), jnp.float32),
        pltpu.VMEM((_CHUNK, _C), jnp.float32),
        pltpu.VMEM((_CHUNK, _C), jnp.float32),
        pltpu.SemaphoreType.DMA,
        pltpu.SemaphoreType.DMA,
        pltpu.SemaphoreType.DMA,
        pltpu.SemaphoreType.DMA,
    ],
)
def _permute_lanes(y_hbm, perm_hbm, out_hbm, perm_v, in0, in1, o0, o1,
                   g0, g1, s0, s1):
    wid = lax.axis_index("s") * _NC + lax.axis_index("c")
    base = wid * _PPW
    pltpu.sync_copy(perm_hbm, perm_v)
    pcs = [perm_v[pl.ds(j * _L, _L)] for j in range(_NBLK)]

    ins = [in0, in1]
    outs = [o0, o1]
    gsems = [g0, g1]
    ssems = [s0, s1]

    def gather_start(i, slot):
        return pltpu.async_copy(
            y_hbm.at[pl.ds(base + i * _CHUNK, _CHUNK)], ins[slot],
            gsems[slot],
        )

    def gather_wait(slot):
        pltpu.make_async_copy(
            y_hbm.at[pl.ds(base, _CHUNK)], ins[slot], gsems[slot]
        ).wait()

    def store_start(i, slot):
        return pltpu.async_copy(
            outs[slot], out_hbm.at[pl.ds(base + i * _CHUNK, _CHUNK)],
            ssems[slot],
        )

    def store_wait(slot):
        pltpu.make_async_copy(
            outs[slot], out_hbm.at[pl.ds(base, _CHUNK)], ssems[slot]
        ).wait()

    def permute_chunk(slot):
        src = ins[slot]
        dst = outs[slot]

        @plsc.parallel_loop(0, _CHUNK, unroll=6)
        def body(p):
            pv = jnp.full((_L,), p, dtype=jnp.int32)
            for j in range(_NBLK):
                dst[p, pl.ds(j * _L, _L)] = plsc.load_gather(
                    src, [pv, pcs[j]]
                )

    gather_start(0, 0)
    gather_start(1, 1)
    for s in (0, 1):
        gather_wait(s)
        permute_chunk(s)
        store_start(s, s)
        gather_start(2 + s, s)

    def pair(t, _):
        k = 2 * t
        for s in (0, 1):
            i = k + s
            gather_wait(s)
            store_wait(s)
            permute_chunk(s)
            store_start(i, s)
            gather_start(i + 2, s)
        return _

    lax.fori_loop(1, _NCHUNK // 2 - 1, pair, None)

    for s in (0, 1):
        gather_wait(s)
        store_wait(s)
        permute_chunk(s)
        store_start(_NCHUNK - 2 + s, s)
    store_wait(0)
    store_wait(1)


def kernel(x, perm):
    y = jnp.transpose(x, (0, 2, 3, 1)).reshape(_P, _C)
    out_t = _permute_lanes(y, perm.astype(jnp.int32))
    return jnp.transpose(out_t.reshape(_B, _H, _W, _C), (0, 3, 1, 2))


# speedup vs baseline: 1.0197x; 1.0197x over previous
import functools

import jax
import jax.numpy as jnp
from jax import lax
from jax.experimental import pallas as pl
from jax.experimental.pallas import tpu as pltpu
from jax.experimental.pallas import tpu_sc as plsc

_B = 32
_C = 192
_H = 56
_W = 56
_P = _B * _H * _W

_info = plsc.get_sparse_core_info()
_NC = _info.num_cores
_NS = _info.num_subcores
_NW = _NC * _NS
_L = _info.num_lanes
_NBLK = _C // _L
_CHUNK = 112
_PPW = _P // _NW
_NCHUNK = _PPW // _CHUNK

_mesh = plsc.VectorSubcoreMesh(core_axis_name="c", subcore_axis_name="s")


@functools.partial(
    pl.kernel,
    mesh=_mesh,
    compiler_params=pltpu.CompilerParams(needs_layout_passes=False),
    out_type=jax.ShapeDtypeStruct((_P, _C), jnp.float32),
    scratch_types=[
        pltpu.VMEM((_C,), jnp.int32),
        pltpu.VMEM((_CHUNK, _C), jnp.float32),
        pltpu.VMEM((_CHUNK, _C), jnp.float32),
        pltpu.VMEM((_CHUNK, _C), jnp.float32),
        pltpu.VMEM((_CHUNK, _C), jnp.float32),
        pltpu.SemaphoreType.DMA,
        pltpu.SemaphoreType.DMA,
        pltpu.SemaphoreType.DMA,
        pltpu.SemaphoreType.DMA,
    ],
)
def _permute_lanes(y_hbm, perm_hbm, out_hbm, perm_v, in0, in1, o0, o1,
                   g0, g1, s0, s1):
    wid = lax.axis_index("s") * _NC + lax.axis_index("c")
    base = wid * _PPW
    pltpu.sync_copy(perm_hbm, perm_v)
    pcs = [perm_v[pl.ds(j * _L, _L)] for j in range(_NBLK)]

    ins = [in0, in1]
    outs = [o0, o1]
    gsems = [g0, g1]
    ssems = [s0, s1]

    def gather_start(i, slot):
        return pltpu.async_copy(
            y_hbm.at[pl.ds(base + i * _CHUNK, _CHUNK)], ins[slot],
            gsems[slot],
        )

    def gather_wait(slot):
        pltpu.make_async_copy(
            y_hbm.at[pl.ds(base, _CHUNK)], ins[slot], gsems[slot]
        ).wait()

    def store_start(i, slot):
        return pltpu.async_copy(
            outs[slot], out_hbm.at[pl.ds(base + i * _CHUNK, _CHUNK)],
            ssems[slot],
        )

    def store_wait(slot):
        pltpu.make_async_copy(
            outs[slot], out_hbm.at[pl.ds(base, _CHUNK)], ssems[slot]
        ).wait()

    def permute_chunk(slot):
        src = ins[slot]
        dst = outs[slot]

        @plsc.parallel_loop(0, _CHUNK, unroll=4)
        def body(p):
            pv = jnp.full((_L,), p, dtype=jnp.int32)
            for j in range(_NBLK):
                dst[p, pl.ds(j * _L, _L)] = plsc.load_gather(
                    src, [pv, pcs[j]]
                )

    gather_start(0, 0)
    gather_start(1, 1)
    for s in (0, 1):
        gather_wait(s)
        permute_chunk(s)
        store_start(s, s)
        gather_start(2 + s, s)

    def pair(t, _):
        k = 2 * t
        for s in (0, 1):
            i = k + s
            gather_wait(s)
            store_wait(s)
            permute_chunk(s)
            store_start(i, s)
            gather_start(i + 2, s)
        return _

    lax.fori_loop(1, _NCHUNK // 2 - 1, pair, None)

    for s in (0, 1):
        gather_wait(s)
        store_wait(s)
        permute_chunk(s)
        store_start(_NCHUNK - 2 + s, s)
    store_wait(0)
    store_wait(1)


def kernel(x, perm):
    y = jnp.transpose(x, (0, 2, 3, 1)).reshape(_P, _C)
    out_t = _permute_lanes(y, perm.astype(jnp.int32))
    return jnp.transpose(out_t.reshape(_B, _H, _W, _C), (0, 3, 1, 2))
